# baseline (speedup 1.0000x reference)
import numpy as np
import jax
import jax.numpy as jnp
from jax import lax
from jax.experimental import pallas as pl
from jax.experimental.pallas import tpu as pltpu
from jax.experimental.pallas import tpu_sc as plsc

_TOPK = 10
_NHEADS = 4
_PADK = 16

def _q_body(qst_ref, wq_ref, bq_ref, q_ref):
    B, C = qst_ref.shape
    dh = C // _NHEADS
    q = lax.dot_general(qst_ref[...], wq_ref[...], (((1,), (1,)), ((), ())),
                        preferred_element_type=jnp.float32)
    q = (q + bq_ref[...]) * np.float32(1.0 / np.sqrt(dh))
    q_ref[...] = q.reshape(B, 1, C)


def _topk_body(q_ref, wk_ref, vis_ref, idx_ref):
    b = pl.program_id(0)
    T = vis_ref.shape[1]
    C = vis_ref.shape[2]
    H = _NHEADS
    dh = C // H
    q = q_ref[0]
    vis = vis_ref[0]
    k = lax.dot_general(vis, wk_ref[...], (((1,), (1,)), ((), ())),
                        preferred_element_type=jnp.float32)
    parts = []
    for h in range(H):
        qh = q[:, h * dh:(h + 1) * dh]
        kh = k[:, h * dh:(h + 1) * dh]
        parts.append(lax.dot_general(qh, kh, (((1,), (1,)), ((), ())),
                                     preferred_element_type=jnp.float32))
    logits = jnp.concatenate(parts, axis=0)
    m = jnp.max(logits, axis=1, keepdims=True)
    e = jnp.exp(logits - m)
    z = jnp.sum(e, axis=1, keepdims=True)
    temp = jnp.sum(e / z, axis=0).reshape(1, T)
    iota_t = lax.broadcasted_iota(jnp.int32, (1, T), 1)
    sel = []
    tw = temp
    for _ in range(_TOPK):
        mval = jnp.max(tw)
        ti = jnp.max(jnp.where(tw == mval, iota_t, jnp.int32(-1)))
        sel.append(ti)
        tw = jnp.where(iota_t == ti, -jnp.inf, tw)
    iota_k = lax.broadcasted_iota(jnp.int32, (1, _PADK), 1)
    selv = jnp.full((1, _PADK), T, jnp.int32)
    for i, ti in enumerate(sel):
        selv = jnp.where(iota_k == i, ti, selv)
    asc = []
    for _ in range(_TOPK):
        mn = jnp.min(selv)
        asc.append(mn)
        selv = jnp.where(selv == mn, jnp.int32(T), selv)
    row = jnp.zeros((1, _PADK), jnp.int32)
    for i in range(_PADK):
        v = asc[i] if i < _TOPK else asc[_TOPK - 1]
        row = jnp.where(iota_k == i, v, row)
    idx_ref[...] = (row + b * T).reshape(1, 1, _PADK)


def _gather_body(audio_hbm, pa_hbm, pv_hbm, idx_hbm,
                 out_a, out_pa, out_pv, idx_v, rows_v, sem):
    info = plsc.get_sparse_core_info()
    nc = info.num_cores
    nw = nc * info.num_subcores
    B = idx_hbm.shape[0]
    wid = lax.axis_index("s") * nc + lax.axis_index("c")
    npairs = 3 * B
    nrounds = (npairs + nw - 1) // nw
    for j in range(nrounds):
        p = wid + nw * j
        bidx = lax.rem(p, B)
        for ti, (tref, oref) in enumerate(
                ((audio_hbm, out_a), (pa_hbm, out_pa), (pv_hbm, out_pv))):
            lo = ti * B

            @pl.when((p >= lo) & (p < lo + B))
            def _(tref=tref, oref=oref, bidx=bidx):
                pltpu.sync_copy(idx_hbm.at[bidx], idx_v)
                pltpu.async_copy(tref.at[idx_v], rows_v, sem).wait()
                pltpu.sync_copy(rows_v, oref.at[bidx])


def kernel(audio_input, visual_input, patch_inputs, qst_input,
           in_proj_weight, in_proj_bias, out_proj_weight, out_proj_bias,
           lin1_w, lin1_b, lin2_w, lin2_b, ln_g, ln_b):
    B, T, C = audio_input.shape
    wq = in_proj_weight[:C]
    wk = in_proj_weight[C:2 * C]
    bq = in_proj_bias[:C].reshape(1, C)

    q = pl.pallas_call(
        _q_body,
        out_shape=jax.ShapeDtypeStruct((B, 1, C), jnp.float32),
    )(qst_input, wq, bq)

    idx = pl.pallas_call(
        _topk_body,
        grid=(B,),
        in_specs=[pl.BlockSpec((1, 1, C), lambda b: (b, 0, 0)),
                  pl.BlockSpec((C, C), lambda b: (0, 0)),
                  pl.BlockSpec((1, T, C), lambda b: (b, 0, 0))],
        out_specs=pl.BlockSpec((1, 1, _PADK), lambda b: (b, 0, 0)),
        out_shape=jax.ShapeDtypeStruct((B, 1, _PADK), jnp.int32),
    )(q, wk, visual_input)
    idx2 = idx.reshape(B, _PADK)

    mesh = plsc.VectorSubcoreMesh(core_axis_name="c", subcore_axis_name="s")
    out_a, out_pa, out_pv = pl.kernel(
        _gather_body,
        mesh=mesh,
        out_type=[jax.ShapeDtypeStruct((B, _PADK, C), jnp.float32)] * 3,
        scratch_types=[pltpu.VMEM((_PADK,), jnp.int32),
                       pltpu.VMEM((_PADK, C), jnp.float32),
                       pltpu.SemaphoreType.DMA],
    )(audio_input.reshape(B * T, C),
      patch_inputs[0].reshape(B * T, C),
      patch_inputs[1].reshape(B * T, C),
      idx2)
    return (out_a[:, :_TOPK, :], out_pa[:, :_TOPK, :], out_pv[:, :_TOPK, :])
